# baseline (speedup 1.0000x reference)
import jax
import jax.numpy as jnp
from jax.experimental import pallas as pl
from jax.experimental.pallas import tpu as pltpu

_BUFFER_SIZE = 4194304
_BUF_LEN = 4195840
_TAIL_ROWS = 3
_XROWS = _BUFFER_SIZE // 512
_OROWS = _BUF_LEN // 512
_R = 512
_GRID = -(-_OROWS // _R)


def _body(tail_ref, a_ref, b_ref, out_ref):
    i = pl.program_id(0)
    top = jnp.where(i == 0, tail_ref[0:3, :], a_ref[5:8, :])
    out_ref[...] = jnp.concatenate([top, b_ref[0 : _R - 3, :]], axis=0)


def kernel(x, buffer):
    x2 = x.reshape(_XROWS, 512)
    buf2 = buffer.reshape(_OROWS, 512)
    out = pl.pallas_call(
        _body,
        grid=(_GRID,),
        out_shape=jax.ShapeDtypeStruct((_OROWS, 512), jnp.float32),
        in_specs=[
            pl.BlockSpec((8, 512), lambda i: (1024, 0)),
            pl.BlockSpec((8, 512), lambda i: (jnp.maximum(64 * i - 1, 0), 0)),
            pl.BlockSpec((_R, 512), lambda i: (jnp.minimum(i, _XROWS // _R - 1), 0)),
        ],
        out_specs=pl.BlockSpec((_R, 512), lambda i: (i, 0)),
    )(buf2, x2, x2)
    return out.reshape(1, _BUF_LEN)


# speedup vs baseline: 8.8203x; 8.8203x over previous
import jax
import jax.numpy as jnp
from jax.experimental import pallas as pl

_BUFFER_SIZE = 4194304
_BUF_LEN = 4195840
_TAIL = _BUF_LEN - _BUFFER_SIZE

_C = 65536
_GRID = -(-_BUF_LEN // _C)
_SIDE = 2048


def _body(tail_ref, a_ref, b_ref, out_ref):
    i = pl.program_id(0)
    top = jnp.where(
        i == 0, tail_ref[:, 0:_TAIL], a_ref[:, _SIDE - _TAIL : _SIDE]
    )
    out_ref[...] = jnp.concatenate([top, b_ref[:, 0 : _C - _TAIL]], axis=1)


def kernel(x, buffer):
    return pl.pallas_call(
        _body,
        grid=(_GRID,),
        out_shape=jax.ShapeDtypeStruct((1, _BUF_LEN), jnp.float32),
        in_specs=[
            pl.BlockSpec((1, _SIDE), lambda i: (0, _BUFFER_SIZE // _SIDE)),
            pl.BlockSpec(
                (1, _SIDE), lambda i: (0, jnp.maximum(i * (_C // _SIDE) - 1, 0))
            ),
            pl.BlockSpec(
                (1, _C), lambda i: (0, jnp.minimum(i, _BUFFER_SIZE // _C - 1))
            ),
        ],
        out_specs=pl.BlockSpec((1, _C), lambda i: (0, i)),
    )(buffer, x, x)


# speedup vs baseline: 13.8478x; 1.5700x over previous
import jax
import jax.numpy as jnp
from jax.experimental import pallas as pl

_BUFFER_SIZE = 4194304
_BUF_LEN = 4195840
_TAIL = _BUF_LEN - _BUFFER_SIZE

_C = 131072
_GRID = -(-_BUF_LEN // _C)
_SIDE = 2048


def _body(tail_ref, a_ref, b_ref, out_ref):
    i = pl.program_id(0)
    top = jnp.where(
        i == 0, tail_ref[:, 0:_TAIL], a_ref[:, _SIDE - _TAIL : _SIDE]
    )
    out_ref[...] = jnp.concatenate([top, b_ref[:, 0 : _C - _TAIL]], axis=1)


def kernel(x, buffer):
    return pl.pallas_call(
        _body,
        grid=(_GRID,),
        out_shape=jax.ShapeDtypeStruct((1, _BUF_LEN), jnp.float32),
        in_specs=[
            pl.BlockSpec((1, _SIDE), lambda i: (0, _BUFFER_SIZE // _SIDE)),
            pl.BlockSpec(
                (1, _SIDE), lambda i: (0, jnp.maximum(i * (_C // _SIDE) - 1, 0))
            ),
            pl.BlockSpec(
                (1, _C), lambda i: (0, jnp.minimum(i, _BUFFER_SIZE // _C - 1))
            ),
        ],
        out_specs=pl.BlockSpec((1, _C), lambda i: (0, i)),
    )(buffer, x, x)


# speedup vs baseline: 18.2818x; 1.3202x over previous
import jax
import jax.numpy as jnp
from jax.experimental import pallas as pl

_BUFFER_SIZE = 4194304
_BUF_LEN = 4195840
_TAIL = _BUF_LEN - _BUFFER_SIZE

_C = 262144
_GRID = -(-_BUF_LEN // _C)
_SIDE = 2048


def _body(tail_ref, a_ref, b_ref, out_ref):
    i = pl.program_id(0)
    top = jnp.where(
        i == 0, tail_ref[:, 0:_TAIL], a_ref[:, _SIDE - _TAIL : _SIDE]
    )
    out_ref[...] = jnp.concatenate([top, b_ref[:, 0 : _C - _TAIL]], axis=1)


def kernel(x, buffer):
    return pl.pallas_call(
        _body,
        grid=(_GRID,),
        out_shape=jax.ShapeDtypeStruct((1, _BUF_LEN), jnp.float32),
        in_specs=[
            pl.BlockSpec((1, _SIDE), lambda i: (0, _BUFFER_SIZE // _SIDE)),
            pl.BlockSpec(
                (1, _SIDE), lambda i: (0, jnp.maximum(i * (_C // _SIDE) - 1, 0))
            ),
            pl.BlockSpec(
                (1, _C), lambda i: (0, jnp.minimum(i, _BUFFER_SIZE // _C - 1))
            ),
        ],
        out_specs=pl.BlockSpec((1, _C), lambda i: (0, i)),
    )(buffer, x, x)


# speedup vs baseline: 23.5525x; 1.2883x over previous
import jax
import jax.numpy as jnp
from jax.experimental import pallas as pl

_BUFFER_SIZE = 4194304
_BUF_LEN = 4195840
_TAIL = _BUF_LEN - _BUFFER_SIZE

_C = 524288
_GRID = -(-_BUF_LEN // _C)
_SIDE = 2048


def _body(tail_ref, a_ref, b_ref, out_ref):
    i = pl.program_id(0)
    top = jnp.where(
        i == 0, tail_ref[:, 0:_TAIL], a_ref[:, _SIDE - _TAIL : _SIDE]
    )
    out_ref[...] = jnp.concatenate([top, b_ref[:, 0 : _C - _TAIL]], axis=1)


def kernel(x, buffer):
    return pl.pallas_call(
        _body,
        grid=(_GRID,),
        out_shape=jax.ShapeDtypeStruct((1, _BUF_LEN), jnp.float32),
        in_specs=[
            pl.BlockSpec((1, _SIDE), lambda i: (0, _BUFFER_SIZE // _SIDE)),
            pl.BlockSpec(
                (1, _SIDE), lambda i: (0, jnp.maximum(i * (_C // _SIDE) - 1, 0))
            ),
            pl.BlockSpec(
                (1, _C), lambda i: (0, jnp.minimum(i, _BUFFER_SIZE // _C - 1))
            ),
        ],
        out_specs=pl.BlockSpec((1, _C), lambda i: (0, i)),
    )(buffer, x, x)


# speedup vs baseline: 25.3316x; 1.0755x over previous
import jax
import jax.numpy as jnp
from jax.experimental import pallas as pl

_BUFFER_SIZE = 4194304
_BUF_LEN = 4195840
_TAIL = _BUF_LEN - _BUFFER_SIZE

_C = 1048576
_GRID = -(-_BUF_LEN // _C)
_SIDE = 2048


def _body(tail_ref, a_ref, b_ref, out_ref):
    i = pl.program_id(0)
    top = jnp.where(
        i == 0, tail_ref[:, 0:_TAIL], a_ref[:, _SIDE - _TAIL : _SIDE]
    )
    out_ref[...] = jnp.concatenate([top, b_ref[:, 0 : _C - _TAIL]], axis=1)


def kernel(x, buffer):
    return pl.pallas_call(
        _body,
        grid=(_GRID,),
        out_shape=jax.ShapeDtypeStruct((1, _BUF_LEN), jnp.float32),
        in_specs=[
            pl.BlockSpec((1, _SIDE), lambda i: (0, _BUFFER_SIZE // _SIDE)),
            pl.BlockSpec(
                (1, _SIDE), lambda i: (0, jnp.maximum(i * (_C // _SIDE) - 1, 0))
            ),
            pl.BlockSpec(
                (1, _C), lambda i: (0, jnp.minimum(i, _BUFFER_SIZE // _C - 1))
            ),
        ],
        out_specs=pl.BlockSpec((1, _C), lambda i: (0, i)),
    )(buffer, x, x)
